# Initial kernel scaffold; baseline (speedup 1.0000x reference)
import jax
import jax.numpy as jnp
from jax.experimental import pallas as pl


def kernel(proc_times, next_op, time_job_ready, job_done, time_ma_ready, pad_mask, op_scheduled, W_ops, W_ma):
    raise NotImplementedError("write your pallas kernel here")


# speedup vs baseline: 1.2364x; 1.2364x over previous
import functools
import math

import jax
import jax.numpy as jnp
from jax import lax
from jax.experimental import pallas as pl

B, J, O, M = 128, 40, 25, 64
D = 256
SCALE = 100.0
JO = J * O
P = 64


def _fused_kernel(pt_ref, no_ref, tjr_ref, jd_ref, tmr_ref, rem_ref,
                  wops_ref, wma_ref, ops_ref, ma_ref, edge_ref):
    f32 = jnp.float32
    pt = pt_ref[0]
    edge_ref[0] = pt * (1.0 / SCALE)

    avg = jnp.sum(pt, axis=1, keepdims=True) * (1.0 / (M * SCALE))
    pos_mask = (pt > 0.0).astype(f32)
    nelig = jnp.sum(pos_mask, axis=1, keepdims=True) * (1.0 / M)

    tjr = tjr_ref[0]
    jd = jd_ref[0]
    sched = jnp.where(jd > 0.0, 0.0, tjr - jnp.min(tjr))
    no = no_ref[0]

    r = lax.broadcasted_iota(jnp.int32, (JO, 1), 0)
    jrow = r // O
    o_row = (r - jrow * O).astype(f32)
    j1h = (lax.broadcasted_iota(jnp.int32, (JO, J), 1) == jrow).astype(f32)
    nr = jnp.dot(j1h, jnp.concatenate([no, sched], axis=1),
                 preferred_element_type=f32)
    no_row = nr[:, 0:1]
    sched_row = nr[:, 1:2]
    opready = jnp.where(o_row == no_row, sched_row, 0.0) * (1.0 / SCALE)

    p_i = lax.broadcasted_iota(jnp.int32, (P, D), 0).astype(f32)
    d_i = lax.broadcasted_iota(jnp.int32, (P, D), 1)
    d_par = (d_i & 1).astype(f32)
    d_even = (d_i - (d_i & 1)).astype(f32)
    ang = p_i * jnp.exp(d_even * (-math.log(10000.0) / D)) + d_par * (math.pi / 2.0)
    pe_tab = jnp.sin(ang)

    pos_row = o_row + no_row
    p1h = (pos_row == lax.broadcasted_iota(jnp.int32, (JO, P), 1).astype(f32)).astype(f32)

    cols = jnp.concatenate([avg, nelig, opready, p1h], axis=1)
    wbig = jnp.concatenate([wops_ref[...], pe_tab], axis=0)
    ops_ref[0] = jnp.dot(cols, wbig, preferred_element_type=f32)

    tmr = tmr_ref[0]
    a_ma = (tmr - jnp.min(tmr)) * (1.0 / SCALE)
    nem = jnp.sum(pos_mask, axis=0, keepdims=True).reshape(M, 1)
    rem = jnp.sum(rem_ref[0])
    frac = nem * (1.0 / (rem + 1e-6))
    ma_ref[0] = jnp.dot(jnp.concatenate([a_ma, frac], axis=1), wma_ref[...],
                        preferred_element_type=f32)


@functools.partial(jax.jit, static_argnames=())
def kernel(proc_times, next_op, time_job_ready, job_done, time_ma_ready,
           pad_mask, op_scheduled, W_ops, W_ma):
    f32 = jnp.float32
    pt = proc_times.reshape(B, JO, M)
    no_col = next_op.astype(f32).reshape(B, J, 1)
    tjr_col = time_job_ready.reshape(B, J, 1)
    jd_col = job_done.astype(f32).reshape(B, J, 1)
    tmr_col = time_ma_ready.reshape(B, M, 1)
    rem_col = jnp.logical_not(jnp.logical_or(pad_mask, op_scheduled)) \
        .astype(f32).reshape(B, JO, 1)
    wopsT = W_ops.T
    wmaT = W_ma.T

    bspec = lambda shape: pl.BlockSpec((1,) + shape, lambda b: (b, 0, 0))
    wspec = lambda shape: pl.BlockSpec(shape, lambda b: (0, 0))

    ops, ma, edge = pl.pallas_call(
        _fused_kernel,
        grid=(B,),
        in_specs=[
            bspec((JO, M)),
            bspec((J, 1)),
            bspec((J, 1)),
            bspec((J, 1)),
            bspec((M, 1)),
            bspec((JO, 1)),
            wspec((3, D)),
            wspec((2, D)),
        ],
        out_specs=[
            bspec((JO, D)),
            bspec((M, D)),
            bspec((JO, M)),
        ],
        out_shape=[
            jax.ShapeDtypeStruct((B, JO, D), f32),
            jax.ShapeDtypeStruct((B, M, D), f32),
            jax.ShapeDtypeStruct((B, JO, M), f32),
        ],
    )(pt, no_col, tjr_col, jd_col, tmr_col, rem_col, wopsT, wmaT)

    return ops.reshape(B, J, O, D), ma, edge
